# Initial kernel scaffold; baseline (speedup 1.0000x reference)
import jax
import jax.numpy as jnp
from jax.experimental import pallas as pl


def kernel(xyz, new_xyz, features):
    raise NotImplementedError("write your pallas kernel here")


# speedup vs baseline: 2.2808x; 2.2808x over previous
import functools

import numpy as np
import jax
import jax.numpy as jnp
from jax import lax
from jax.experimental import pallas as pl
from jax.experimental.pallas import tpu as pltpu
from jax.experimental.pallas import tpu_sc as plsc

_RADIUS = 0.2
_NSAMPLE = 32
_R2 = np.float32(_RADIUS * _RADIUS)


def _ball_query_idx(xyz, new_xyz):
    B, N, _ = xyz.shape
    d2 = (jnp.sum(new_xyz * new_xyz, axis=-1)[:, :, None]
          + jnp.sum(xyz * xyz, axis=-1)[:, None, :]
          - 2.0 * jnp.einsum('bpd,bnd->bpn', new_xyz, xyz))
    mask = d2 < (_RADIUS * _RADIUS)
    ar = jnp.arange(N, dtype=jnp.int32)
    keyv = jnp.where(mask, ar[None, None, :], jnp.int32(N))
    neg_top, _ = jax.lax.top_k(-keyv, _NSAMPLE)
    idx_sorted = -neg_top
    cnt = jnp.minimum(jnp.sum(mask, axis=-1), _NSAMPLE)
    first = idx_sorted[..., :1]
    slot = jnp.arange(_NSAMPLE, dtype=jnp.int32)
    idx = jnp.where(slot[None, None, :] < cnt[..., None], idx_sorted, first)
    idx = jnp.where(cnt[..., None] > 0, idx, 0)
    return idx.astype(jnp.int32)


_PB = 256


def _ball_query_body(q_ref, xt_ref, o_ref, *, N):
    b = pl.program_id(0)
    q = q_ref[0]
    xt = xt_ref[0]
    NH = N // 16

    q0, q1, q2 = q[:, 0:1], q[:, 1:2], q[:, 2:3]
    x0, x1, x2 = xt[0:1, :], xt[1:2, :], xt[2:3, :]
    sq = (q0 * q0 + q1 * q1) + q2 * q2
    sx = (x0 * x0 + x1 * x1) + x2 * x2
    qx = (q0 * x0 + q1 * x1) + q2 * x2
    d2 = (sq + sx) - 2.0 * qx
    mb = (d2 < _R2).astype(jnp.bfloat16)

    n_i = lax.broadcasted_iota(jnp.int32, (N, NH), 0)
    h_i = lax.broadcasted_iota(jnp.int32, (N, NH), 1)
    blk = (n_i // 16) == h_i
    pw2 = jnp.where(blk, (1 << (n_i % 16)).astype(jnp.float32), 0.0)
    w_pack = pw2.astype(jnp.bfloat16)
    w_cnt = blk.astype(jnp.bfloat16)
    dn = (((1,), (0,)), ((), ()))
    pk = lax.dot_general(mb, w_pack, dn,
                         preferred_element_type=jnp.float32)
    cn = lax.dot_general(mb, w_cnt, dn,
                         preferred_element_type=jnp.float32)

    a_i = lax.broadcasted_iota(jnp.int32, (NH, NH), 0)
    b_i = lax.broadcasted_iota(jnp.int32, (NH, NH), 1)
    tri = (a_i < b_i).astype(jnp.bfloat16)
    ce = lax.dot_general(cn.astype(jnp.bfloat16), tri, dn,
                         preferred_element_type=jnp.float32)
    ci = ce + cn
    cnt = ci[:, NH - 1:NH]

    hv = lax.broadcasted_iota(jnp.float32, (1, NH), 1)
    cols = []
    for s in range(_NSAMPLE):
        sf = jnp.float32(s)
        onehot = jnp.where((ce <= sf) & (ci > sf), 1.0, 0.0)
        h_s = jnp.sum(onehot * hv, axis=1, keepdims=True)
        c_at = jnp.sum(onehot * ce, axis=1, keepdims=True)
        v_at = jnp.sum(onehot * pk, axis=1, keepdims=True)
        cols.append((h_s, c_at, v_at))
    h_s = jnp.concatenate([c[0] for c in cols], axis=1)
    c_at = jnp.concatenate([c[1] for c in cols], axis=1)
    v_at = jnp.concatenate([c[2] for c in cols], axis=1)
    j_s = lax.broadcasted_iota(jnp.float32, (1, _NSAMPLE), 1) - c_at

    u = v_at
    pp = jnp.zeros_like(v_at)
    bitpos = jnp.zeros_like(v_at)
    for _ in range(16):
        un = jnp.floor(u * 0.5)
        pp = pp + (u - 2.0 * un)
        bitpos = bitpos + jnp.where(pp <= j_s, 1.0, 0.0)
        u = un
    idxf = h_s * 16.0 + bitpos

    slot = lax.broadcasted_iota(jnp.float32, (1, _NSAMPLE), 1)
    idxf = jnp.where(slot < cnt, idxf, idxf[:, 0:1])
    idxf = jnp.where(cnt > 0.0, idxf, 0.0)
    o_ref[0] = idxf.astype(jnp.int32) + b * N


def _ball_query_pallas(xyz, new_xyz):
    B, N, _ = xyz.shape
    P = new_xyz.shape[1]
    xt = jnp.transpose(xyz, (0, 2, 1))
    body = functools.partial(_ball_query_body, N=N)
    return pl.pallas_call(
        body,
        grid=(B, P // _PB),
        in_specs=[
            pl.BlockSpec((1, _PB, 3), lambda b, i: (b, i, 0)),
            pl.BlockSpec((1, 3, N), lambda b, i: (b, 0, 0)),
        ],
        out_specs=pl.BlockSpec((1, _PB, _NSAMPLE), lambda b, i: (b, i, 0)),
        out_shape=jax.ShapeDtypeStruct((B, P, _NSAMPLE), jnp.int32),
    )(new_xyz, xt)


def _sc_gather(table, flat_idx):
    R, D = table.shape
    M = flat_idx.shape[0]
    W = 128
    mesh = plsc.VectorSubcoreMesh(core_axis_name="c", subcore_axis_name="s")
    idx2 = flat_idx.reshape(1, M)

    @functools.partial(
        pl.kernel,
        out_type=jax.ShapeDtypeStruct((M, D), table.dtype),
        mesh=mesh,
    )
    def k(tab_hbm, i_hbm, o_hbm):
        def body(i_vmem, o_vmem):
            pltpu.sync_copy(tab_hbm.at[i_vmem.at[0]], o_vmem)

        pltpu.emit_pipeline(
            body,
            grid=(M // W,),
            in_specs=[pl.BlockSpec((1, W), lambda i: (0, i))],
            out_specs=[pl.BlockSpec((W, D), lambda i: (i, 0))],
            core_axis_name=("c", "s"),
            dimension_semantics=(pltpu.PARALLEL,),
        )(i_hbm, o_hbm)

    return k(table, idx2)


def _finalize_body(g_ref, q_ref, o_ref, *, C):
    g = g_ref[0]
    t = jnp.swapaxes(g, 0, 1)
    o_ref[0, 0:3] = t[C:C + 3] - q_ref[0]
    o_ref[0, 3:3 + C] = t[0:C]


def _finalize(gathered, qrep_t, C):
    B, _, PS = qrep_t.shape
    D = gathered.shape[-1]
    Pb = 128
    Mb = Pb * _NSAMPLE
    body = functools.partial(_finalize_body, C=C)
    out = pl.pallas_call(
        body,
        grid=(B, PS // Mb),
        in_specs=[
            pl.BlockSpec((1, Mb, D), lambda b, i: (b, i, 0)),
            pl.BlockSpec((1, 3, Mb), lambda b, i: (b, 0, i)),
        ],
        out_specs=pl.BlockSpec((1, 3 + C, Mb), lambda b, i: (b, 0, i)),
        out_shape=jax.ShapeDtypeStruct((B, 3 + C, PS), jnp.float32),
    )(gathered, qrep_t)
    return out


def kernel(xyz, new_xyz, features):
    B, N, _ = xyz.shape
    P = new_xyz.shape[1]
    C = features.shape[2]

    idx = _ball_query_idx(xyz, new_xyz)
    flat_idx = (idx + (jnp.arange(B, dtype=jnp.int32) * N)[:, None, None])
    flat_idx = flat_idx.reshape(-1)

    D = 128
    pad = jnp.zeros((B, N, D - C - 3), dtype=jnp.float32)
    table = jnp.concatenate([features, xyz, pad], axis=-1)
    table = table.reshape(B * N, D)

    gathered = _sc_gather(table, flat_idx)
    qrep_t = jnp.repeat(jnp.transpose(new_xyz, (0, 2, 1)), _NSAMPLE, axis=2)
    out = _finalize(gathered.reshape(B, P * _NSAMPLE, D), qrep_t, C)
    return out.reshape(B, 3 + C, P, _NSAMPLE)


# speedup vs baseline: 22.2576x; 9.7587x over previous
import functools

import numpy as np
import jax
import jax.numpy as jnp
from jax import lax
from jax.experimental import pallas as pl
from jax.experimental.pallas import tpu as pltpu
from jax.experimental.pallas import tpu_sc as plsc

_RADIUS = 0.2
_NSAMPLE = 32
_R2 = np.float32(_RADIUS * _RADIUS)


def _ball_query_idx(xyz, new_xyz):
    B, N, _ = xyz.shape
    d2 = (jnp.sum(new_xyz * new_xyz, axis=-1)[:, :, None]
          + jnp.sum(xyz * xyz, axis=-1)[:, None, :]
          - 2.0 * jnp.einsum('bpd,bnd->bpn', new_xyz, xyz))
    mask = d2 < (_RADIUS * _RADIUS)
    ar = jnp.arange(N, dtype=jnp.int32)
    keyv = jnp.where(mask, ar[None, None, :], jnp.int32(N))
    neg_top, _ = jax.lax.top_k(-keyv, _NSAMPLE)
    idx_sorted = -neg_top
    cnt = jnp.minimum(jnp.sum(mask, axis=-1), _NSAMPLE)
    first = idx_sorted[..., :1]
    slot = jnp.arange(_NSAMPLE, dtype=jnp.int32)
    idx = jnp.where(slot[None, None, :] < cnt[..., None], idx_sorted, first)
    idx = jnp.where(cnt[..., None] > 0, idx, 0)
    return idx.astype(jnp.int32)


_PB = 256


def _ball_query_body(q_ref, xt_ref, o_ref, *, N):
    b = pl.program_id(0)
    q = q_ref[0]
    xt = xt_ref[0]
    NH = N // 16

    q0, q1, q2 = q[:, 0:1], q[:, 1:2], q[:, 2:3]
    x0, x1, x2 = xt[0:1, :], xt[1:2, :], xt[2:3, :]
    sq = (q0 * q0 + q1 * q1) + q2 * q2
    sx = (x0 * x0 + x1 * x1) + x2 * x2
    qx = lax.dot_general(q.astype(jnp.bfloat16), xt.astype(jnp.bfloat16),
                         (((1,), (0,)), ((), ())),
                         preferred_element_type=jnp.float32)
    d2 = (sq + sx) - 2.0 * qx
    mb = (d2 < _R2).astype(jnp.bfloat16)

    n_i = lax.broadcasted_iota(jnp.int32, (N, NH), 0)
    h_i = lax.broadcasted_iota(jnp.int32, (N, NH), 1)
    blk = (n_i // 16) == h_i
    pw2 = jnp.where(blk, (1 << (n_i % 16)).astype(jnp.float32), 0.0)
    w_pack = pw2.astype(jnp.bfloat16)
    w_cnt = blk.astype(jnp.bfloat16)
    dn = (((1,), (0,)), ((), ()))
    pk = lax.dot_general(mb, w_pack, dn,
                         preferred_element_type=jnp.float32)
    cn = lax.dot_general(mb, w_cnt, dn,
                         preferred_element_type=jnp.float32)

    a_i = lax.broadcasted_iota(jnp.int32, (NH, NH), 0)
    b_i = lax.broadcasted_iota(jnp.int32, (NH, NH), 1)
    tri = (a_i < b_i).astype(jnp.bfloat16)
    ce = lax.dot_general(cn.astype(jnp.bfloat16), tri, dn,
                         preferred_element_type=jnp.float32)
    ci = ce + cn
    cnt = ci[:, NH - 1:NH]

    hv = lax.broadcasted_iota(jnp.int32, (1, NH), 1).astype(jnp.float32)
    cols = []
    for s in range(_NSAMPLE):
        sf = jnp.float32(s)
        onehot = jnp.where((ce <= sf) & (ci > sf), 1.0, 0.0)
        h_s = jnp.sum(onehot * hv, axis=1, keepdims=True)
        c_at = jnp.sum(onehot * ce, axis=1, keepdims=True)
        v_at = jnp.sum(onehot * pk, axis=1, keepdims=True)
        cols.append((h_s, c_at, v_at))
    h_s = jnp.concatenate([c[0] for c in cols], axis=1)
    c_at = jnp.concatenate([c[1] for c in cols], axis=1)
    v_at = jnp.concatenate([c[2] for c in cols], axis=1)
    j_s = lax.broadcasted_iota(jnp.int32, (1, _NSAMPLE), 1).astype(jnp.float32) - c_at

    u = v_at
    pp = jnp.zeros_like(v_at)
    bitpos = jnp.zeros_like(v_at)
    for _ in range(16):
        un = jnp.floor(u * 0.5)
        pp = pp + (u - 2.0 * un)
        bitpos = bitpos + jnp.where(pp <= j_s, 1.0, 0.0)
        u = un
    idxf = h_s * 16.0 + bitpos

    slot = lax.broadcasted_iota(jnp.int32, (1, _NSAMPLE), 1).astype(jnp.float32)
    idxf = jnp.where(slot < cnt, idxf, idxf[:, 0:1])
    idxf = jnp.where(cnt > 0.0, idxf, 0.0)
    o_ref[0] = idxf.astype(jnp.int32) + b * N


def _ball_query_pallas(xyz, new_xyz):
    B, N, _ = xyz.shape
    P = new_xyz.shape[1]
    xt = jnp.transpose(xyz, (0, 2, 1))
    body = functools.partial(_ball_query_body, N=N)
    return pl.pallas_call(
        body,
        grid=(B, P // _PB),
        in_specs=[
            pl.BlockSpec((1, _PB, 3), lambda b, i: (b, i, 0)),
            pl.BlockSpec((1, 3, N), lambda b, i: (b, 0, 0)),
        ],
        out_specs=pl.BlockSpec((1, _PB, _NSAMPLE), lambda b, i: (b, i, 0)),
        out_shape=jax.ShapeDtypeStruct((B, P, _NSAMPLE), jnp.int32),
    )(new_xyz, xt)


def _sc_gather(table, flat_idx):
    R, D = table.shape
    M = flat_idx.shape[0]
    W = 128
    mesh = plsc.VectorSubcoreMesh(core_axis_name="c", subcore_axis_name="s")
    idx2 = flat_idx.reshape(1, M)

    @functools.partial(
        pl.kernel,
        out_type=jax.ShapeDtypeStruct((M, D), table.dtype),
        mesh=mesh,
    )
    def k(tab_hbm, i_hbm, o_hbm):
        def body(i_vmem, o_vmem):
            pltpu.sync_copy(tab_hbm.at[i_vmem.at[0]], o_vmem)

        pltpu.emit_pipeline(
            body,
            grid=(M // W,),
            in_specs=[pl.BlockSpec((1, W), lambda i: (0, i))],
            out_specs=[pl.BlockSpec((W, D), lambda i: (i, 0))],
            core_axis_name=("c", "s"),
            dimension_semantics=(pltpu.PARALLEL,),
        )(i_hbm, o_hbm)

    return k(table, idx2)


def _finalize_body(g_ref, q_ref, o_ref, *, C):
    g = g_ref[0]
    t = jnp.swapaxes(g, 0, 1)
    o_ref[0, 0:3] = t[C:C + 3] - q_ref[0]
    o_ref[0, 3:3 + C] = t[0:C]


def _finalize(gathered, qrep_t, C):
    B, _, PS = qrep_t.shape
    D = gathered.shape[-1]
    Pb = 128
    Mb = Pb * _NSAMPLE
    body = functools.partial(_finalize_body, C=C)
    out = pl.pallas_call(
        body,
        grid=(B, PS // Mb),
        in_specs=[
            pl.BlockSpec((1, Mb, D), lambda b, i: (b, i, 0)),
            pl.BlockSpec((1, 3, Mb), lambda b, i: (b, 0, i)),
        ],
        out_specs=pl.BlockSpec((1, 3 + C, Mb), lambda b, i: (b, 0, i)),
        out_shape=jax.ShapeDtypeStruct((B, 3 + C, PS), jnp.float32),
    )(gathered, qrep_t)
    return out


def kernel(xyz, new_xyz, features):
    B, N, _ = xyz.shape
    P = new_xyz.shape[1]
    C = features.shape[2]

    flat_idx = _ball_query_pallas(xyz, new_xyz).reshape(-1)

    D = 128
    pad = jnp.zeros((B, N, D - C - 3), dtype=jnp.float32)
    table = jnp.concatenate([features, xyz, pad], axis=-1)
    table = table.reshape(B * N, D)

    gathered = _sc_gather(table, flat_idx)
    qrep_t = jnp.repeat(jnp.transpose(new_xyz, (0, 2, 1)), _NSAMPLE, axis=2)
    out = _finalize(gathered.reshape(B, P * _NSAMPLE, D), qrep_t, C)
    return out.reshape(B, 3 + C, P, _NSAMPLE)
